# baseline (speedup 1.0000x reference)
import jax
import jax.numpy as jnp
from jax import lax
from jax.experimental import pallas as pl
from jax.experimental.pallas import tpu as pltpu
from jax.experimental.pallas import tpu_sc as plsc

N = 10000
E = 160000
F_IN = 256
H = 512
C = 10
G = 64

NP = 10240
RT = NP // 16
NTILE = 16
EB = 128
NB = 80
E_PAD = NTILE * NB * EB
EB2 = 64
NB2 = NB * 2
NP_ACC = 10048
NCHUNK = 4
CW = H // NCHUNK
PASSES = NCHUNK // 2
NROWB = 10
RB = NP // NROWB

_f32 = jnp.float32
_i32 = jnp.int32
_i16 = jnp.int16

_sc_mesh = plsc.VectorSubcoreMesh(core_axis_name="c", subcore_axis_name="s")


def _widen_idx(pk_h, pk_ref, idx32_ref):
    mask = jnp.full((16,), 0xFFFF, _i32)
    pltpu.sync_copy(pk_h, pk_ref)

    def body(j, carry):
        for k in range(EB // 16):
            v = pk_ref[j, pl.ds(k * 16, 16)]
            idx32_ref[2 * j, pl.ds(k * 16, 16)] = lax.bitwise_and(v, mask)
            idx32_ref[2 * j + 1, pl.ds(k * 16, 16)] = (
                lax.shift_right_logical(v, 16))
        return carry

    lax.fori_loop(0, NB // 2, body, 0)


def _fill_buf(zb_ref, nrows, val):
    v16 = jnp.full((16,), val, _f32)

    def body(r, carry):
        for k in range(CW // 16):
            zb_ref[r, pl.ds(k * 16, 16)] = v16
        return carry

    lax.fori_loop(0, nrows, body, 0)


def _deg_body(dst_h, out_h, d16, dst_v, ones_v, gb, acc):
    c = lax.axis_index("c")
    s = lax.axis_index("s")
    _widen_idx(dst_h.at[s], d16, dst_v)
    _fill_buf(ones_v, EB2, 1.0)
    _fill_buf(gb, EB2, 0.0)
    for k in range(RT // EB2):
        pltpu.sync_copy(gb, acc.at[pl.ds(s * RT + k * EB2, EB2)])
    plsc.subcore_barrier()

    def body(j, carry):
        b = c * NB + j
        pltpu.sync_copy(
            ones_v, acc.at[dst_v.at[b // 2, pl.ds((b % 2) * EB2, EB2)]],
            add=True)
        return carry

    lax.fori_loop(0, NB, body, 0)
    plsc.subcore_barrier()
    for k in range(RT // EB2):
        pltpu.sync_copy(acc.at[pl.ds(s * RT + k * EB2, EB2)], gb)
        pltpu.sync_copy(gb, out_h.at[c, pl.ds(s * RT + k * EB2, EB2)])


def _deg_kernel(dst16_t):
    fn = pl.kernel(
        _deg_body,
        out_type=jax.ShapeDtypeStruct((2, NP, EB), _f32),
        mesh=_sc_mesh,
        scratch_types=[
            pltpu.VMEM((NB // 2, EB), _i32),
            pltpu.VMEM((NB, EB), _i32),
            pltpu.VMEM((EB2, EB), _f32),
            pltpu.VMEM((EB2, EB), _f32),
            pltpu.VMEM_SHARED((NP, EB), _f32),
        ],
    )
    return fn(dst16_t)


def _scatter_body(*refs):
    ys = refs[:NCHUNK]
    src_h, dst_h = refs[NCHUNK:NCHUNK + 2]
    zs = refs[NCHUNK + 2:2 * NCHUNK + 2]
    (s16, src_v, dst_v, gb0, gb1, acc,
     sem0, sem1, sems0, sems1) = refs[2 * NCHUNK + 2:]
    c = lax.axis_index("c")
    s = lax.axis_index("s")
    _widen_idx(src_h.at[s], s16, src_v)
    _widen_idx(dst_h.at[s], s16, dst_v)

    for p in range(PASSES):
        _fill_buf(gb0, EB2, 0.0)
        for k in range(RT // EB2):
            @pl.when(s * RT + (k + 1) * EB2 <= NP_ACC)
            def _(k=k):
                pltpu.sync_copy(gb0, acc.at[pl.ds(s * RT + k * EB2, EB2)])

        for cc in range(2):
            yc = ys[cc * PASSES + p]

            @pl.when(c == cc)
            def _(yc=yc):
                pltpu.async_copy(yc.at[src_v.at[0, pl.ds(0, EB2)]], gb0, sem0)
                pltpu.async_copy(yc.at[src_v.at[0, pl.ds(EB2, EB2)]], gb1, sem1)

        plsc.subcore_barrier()

        for cc in range(2):
            yc = ys[cc * PASSES + p]

            @pl.when(c == cc)
            def _(yc=yc):
                lo = pl.ds(0, EB2)
                hi = pl.ds(EB2, EB2)

                def body(jj, carry):
                    pltpu.make_async_copy(
                        yc.at[src_v.at[jj, lo]], gb0, sem0).wait()
                    pltpu.async_copy(
                        gb0, acc.at[dst_v.at[jj, lo]], sems0, add=True)
                    pltpu.make_async_copy(
                        yc.at[src_v.at[jj, hi]], gb1, sem1).wait()
                    pltpu.async_copy(
                        gb1, acc.at[dst_v.at[jj, hi]], sems1, add=True)
                    pltpu.make_async_copy(
                        gb0, acc.at[dst_v.at[jj, lo]], sems0).wait()

                    @pl.when(jj < NB - 1)
                    def _():
                        pltpu.async_copy(
                            yc.at[src_v.at[jj + 1, lo]], gb0, sem0)

                    pltpu.make_async_copy(
                        gb1, acc.at[dst_v.at[jj, hi]], sems1).wait()

                    @pl.when(jj < NB - 1)
                    def _():
                        pltpu.async_copy(
                            yc.at[src_v.at[jj + 1, hi]], gb1, sem1)
                    return carry

                lax.fori_loop(0, NB, body, 0)

        plsc.subcore_barrier()

        for cc in range(2):
            zc = zs[cc * PASSES + p]

            @pl.when(c == cc)
            def _(zc=zc):
                for k in range(RT // EB2):
                    @pl.when(s * RT + (k + 1) * EB2 <= NP_ACC)
                    def _(k=k):
                        pltpu.sync_copy(acc.at[pl.ds(s * RT + k * EB2, EB2)], gb0)
                        pltpu.sync_copy(gb0, zc.at[pl.ds(s * RT + k * EB2, EB2)])

        if p != PASSES - 1:
            plsc.subcore_barrier()


def _sc_scatter(ychunks, src16_t, dst16_t):
    fn = pl.kernel(
        _scatter_body,
        out_type=[jax.ShapeDtypeStruct((NP, CW), _f32)] * NCHUNK,
        mesh=_sc_mesh,
        scratch_types=[
            pltpu.VMEM((NB // 2, EB), _i32),
            pltpu.VMEM((NB, EB), _i32),
            pltpu.VMEM((NB, EB), _i32),
            pltpu.VMEM((EB2, CW), _f32),
            pltpu.VMEM((EB2, CW), _f32),
            pltpu.VMEM_SHARED((NP_ACC, CW), _f32),
            pltpu.SemaphoreType.DMA,
            pltpu.SemaphoreType.DMA,
            pltpu.SemaphoreType.DMA,
            pltpu.SemaphoreType.DMA,
        ],
    )
    return fn(*ychunks, src16_t, dst16_t)


def _dinv_block(dg_ref):
    d = dg_ref[0][:, 0:1] + dg_ref[1][:, 0:1]
    return lax.rsqrt(1.0 + d)


def _mm_scale_body(x_ref, w_ref, dg_ref, *outs):
    dinv = _dinv_block(dg_ref)
    y = jnp.dot(x_ref[...] * dinv, w_ref[...], preferred_element_type=_f32)
    for cidx, o in enumerate(outs):
        o[...] = y[:, cidx * CW:(cidx + 1) * CW]


def _mm_scale(x_p, W1, degpart):
    return pl.pallas_call(
        _mm_scale_body,
        grid=(NROWB,),
        in_specs=[
            pl.BlockSpec((RB, F_IN), lambda i: (i, 0)),
            pl.BlockSpec((F_IN, H), lambda i: (0, 0)),
            pl.BlockSpec((2, RB, EB), lambda i: (0, i, 0)),
        ],
        out_specs=[pl.BlockSpec((RB, CW), lambda i: (i, 0))] * NCHUNK,
        out_shape=[jax.ShapeDtypeStruct((NP, CW), _f32)] * NCHUNK,
    )(x_p, W1, degpart)


def _mid_body(*refs):
    zs = refs[:NCHUNK]
    ys = refs[NCHUNK:2 * NCHUNK]
    dg_ref, b_ref, w_ref = refs[2 * NCHUNK:2 * NCHUNK + 3]
    outs = refs[2 * NCHUNK + 3:]
    dinv = _dinv_block(dg_ref)
    hs = jnp.concatenate([zr[...] + yr[...] for zr, yr in zip(zs, ys)], axis=1)
    t = dinv * jax.nn.relu(dinv * hs + b_ref[...])
    out = jnp.dot(t, w_ref[...], preferred_element_type=_f32)
    for cidx, o in enumerate(outs):
        o[...] = out[:, cidx * CW:(cidx + 1) * CW]


def _mid_layer(zchunks, ychunks, degpart, b1r, W2):
    return pl.pallas_call(
        _mid_body,
        grid=(NROWB,),
        in_specs=(
            [pl.BlockSpec((RB, CW), lambda i: (i, 0))] * (2 * NCHUNK)
            + [
                pl.BlockSpec((2, RB, EB), lambda i: (0, i, 0)),
                pl.BlockSpec((1, H), lambda i: (0, 0)),
                pl.BlockSpec((H, H), lambda i: (0, 0)),
            ]
        ),
        out_specs=[pl.BlockSpec((RB, CW), lambda i: (i, 0))] * NCHUNK,
        out_shape=[jax.ShapeDtypeStruct((NP, CW), _f32)] * NCHUNK,
    )(*zchunks, *ychunks, degpart, b1r, W2)


def _final_body(*refs):
    zs = refs[:NCHUNK]
    ys = refs[NCHUNK:2 * NCHUNK]
    dg_ref, b_ref, bt_ref, wl_ref, bl_ref = refs[2 * NCHUNK:2 * NCHUNK + 5]
    hg_ref, lp_ref, cnt_ref = refs[2 * NCHUNK + 5:]
    i = pl.program_id(0)

    @pl.when(i == 0)
    def _():
        hg_ref[...] = jnp.zeros_like(hg_ref)
        cnt_ref[...] = jnp.zeros_like(cnt_ref)
        lp_ref[...] = jnp.zeros_like(lp_ref)

    dinv = _dinv_block(dg_ref)
    hs = jnp.concatenate([zr[...] + yr[...] for zr, yr in zip(zs, ys)], axis=1)
    bt = bt_ref[...]
    h = jax.nn.relu(dinv * hs + b_ref[...])
    h = jnp.where(bt < G, h, 0.0)
    gid = lax.broadcasted_iota(_i32, (1, G), 1)
    oh = (bt == gid).astype(_f32)
    dims = (((0,), (0,)), ((), ()))
    hg_ref[...] += lax.dot_general(oh, h, dims, preferred_element_type=_f32)
    cnt_ref[...] += lax.dot_general(oh, jnp.ones((RB, 128), _f32), dims,
                                    preferred_element_type=_f32)

    @pl.when(i == NROWB - 1)
    def _():
        cnt = jnp.maximum(cnt_ref[:, 0:1], 1.0)
        hg = hg_ref[...] / cnt
        hg_ref[...] = hg
        logits = jnp.dot(hg, wl_ref[...], preferred_element_type=_f32) + bl_ref[...]
        m = jnp.max(logits, axis=1, keepdims=True)
        lse = jnp.log(jnp.sum(jnp.exp(logits - m), axis=1, keepdims=True)) + m
        lp_ref[...] = logits - lse


def _final(zchunks, ychunks, degpart, b2r, batch_p, Wl, blr):
    return pl.pallas_call(
        _final_body,
        grid=(NROWB,),
        in_specs=(
            [pl.BlockSpec((RB, CW), lambda i: (i, 0))] * (2 * NCHUNK)
            + [
                pl.BlockSpec((2, RB, EB), lambda i: (0, i, 0)),
                pl.BlockSpec((1, H), lambda i: (0, 0)),
                pl.BlockSpec((RB, 1), lambda i: (i, 0)),
                pl.BlockSpec((H, C), lambda i: (0, 0)),
                pl.BlockSpec((1, C), lambda i: (0, 0)),
            ]
        ),
        out_specs=[
            pl.BlockSpec((G, H), lambda i: (0, 0)),
            pl.BlockSpec((G, C), lambda i: (0, 0)),
        ],
        out_shape=[
            jax.ShapeDtypeStruct((G, H), _f32),
            jax.ShapeDtypeStruct((G, C), _f32),
        ],
        scratch_shapes=[pltpu.VMEM((G, 128), _f32)],
        compiler_params=pltpu.CompilerParams(
            dimension_semantics=("arbitrary",)),
    )(*zchunks, *ychunks, degpart, b2r, batch_p, Wl, blr)


@jax.jit
def kernel(x, edge_index, batch, W1, b1, W2, b2, Wl, bl):
    src = edge_index[0].astype(_i32)
    dst = edge_index[1].astype(_i32)
    pad = jnp.full((E_PAD - E,), N, _i32)
    def _pack(a):
        a = jnp.concatenate([a, pad]).reshape(NTILE, NB // 2, 2, EB)
        return a[:, :, 0, :] | (a[:, :, 1, :] << 16)

    src16_t = _pack(src)
    dst16_t = _pack(dst)
    x_p = jnp.pad(x, ((0, NP - N), (0, 0)))
    batch_p = jnp.concatenate(
        [batch.astype(_i32), jnp.full((NP - N,), G, _i32)]).reshape(NP, 1)
    degpart = _deg_kernel(dst16_t)
    y1 = _mm_scale(x_p, W1, degpart)
    zc1 = _sc_scatter(y1, src16_t, dst16_t)
    y2 = _mid_layer(zc1, y1, degpart, b1.reshape(1, H), W2)
    zc2 = _sc_scatter(y2, src16_t, dst16_t)
    hG, logp = _final(zc2, y2, degpart, b2.reshape(1, H), batch_p,
                      Wl, bl.reshape(1, C))
    return (hG, logp)


# speedup vs baseline: 1.1255x; 1.1255x over previous
import jax
import jax.numpy as jnp
from jax import lax
from jax.experimental import pallas as pl
from jax.experimental.pallas import tpu as pltpu
from jax.experimental.pallas import tpu_sc as plsc

N = 10000
E = 160000
F_IN = 256
H = 512
C = 10
G = 64

NP = 10240
RT = NP // 16
NTILE = 16
EB = 128
NB = 80
E_PAD = NTILE * NB * EB
EB2 = 64
NB2 = NB * 2
NP_ACC = 10048
NCHUNK = 4
CW = H // NCHUNK
PASSES = NCHUNK // 2
NROWB = 10
RB = NP // NROWB

_f32 = jnp.float32
_i32 = jnp.int32
_i16 = jnp.int16

_sc_mesh = plsc.VectorSubcoreMesh(core_axis_name="c", subcore_axis_name="s")


def _widen_idx(pk_h, pk_ref, idx32_ref):
    mask = jnp.full((16,), 0xFFFF, _i32)
    pltpu.sync_copy(pk_h, pk_ref)

    def body(j, carry):
        for k in range(EB // 16):
            v = pk_ref[j, pl.ds(k * 16, 16)]
            idx32_ref[2 * j, pl.ds(k * 16, 16)] = lax.bitwise_and(v, mask)
            idx32_ref[2 * j + 1, pl.ds(k * 16, 16)] = (
                lax.shift_right_logical(v, 16))
        return carry

    lax.fori_loop(0, NB // 2, body, 0)


def _fill_buf(zb_ref, nrows, val):
    v16 = jnp.full((16,), val, _f32)

    def body(r, carry):
        for k in range(CW // 16):
            zb_ref[r, pl.ds(k * 16, 16)] = v16
        return carry

    lax.fori_loop(0, nrows, body, 0)


def _deg_body(dst_h, src_h, out_h, srcw_h, dstw_h, d16, dst_v, src_v,
              ones_v, gb, acc):
    c = lax.axis_index("c")
    s = lax.axis_index("s")
    _widen_idx(dst_h.at[s], d16, dst_v)
    pltpu.sync_copy(dst_v, dstw_h.at[s])
    _widen_idx(src_h.at[s], d16, src_v)
    pltpu.sync_copy(src_v, srcw_h.at[s])
    _fill_buf(ones_v, EB2, 1.0)
    _fill_buf(gb, EB2, 0.0)
    for k in range(RT // EB2):
        pltpu.sync_copy(gb, acc.at[pl.ds(s * RT + k * EB2, EB2)])
    plsc.subcore_barrier()

    def body(j, carry):
        b = c * NB + j
        pltpu.sync_copy(
            ones_v, acc.at[dst_v.at[b // 2, pl.ds((b % 2) * EB2, EB2)]],
            add=True)
        return carry

    lax.fori_loop(0, NB, body, 0)
    plsc.subcore_barrier()
    for k in range(RT // EB2):
        pltpu.sync_copy(acc.at[pl.ds(s * RT + k * EB2, EB2)], gb)
        pltpu.sync_copy(gb, out_h.at[c, pl.ds(s * RT + k * EB2, EB2)])


def _deg_kernel(dst16_t, src16_t):
    fn = pl.kernel(
        _deg_body,
        out_type=[
            jax.ShapeDtypeStruct((2, NP, EB), _f32),
            jax.ShapeDtypeStruct((NTILE, NB, EB), _i32),
            jax.ShapeDtypeStruct((NTILE, NB, EB), _i32),
        ],
        mesh=_sc_mesh,
        scratch_types=[
            pltpu.VMEM((NB // 2, EB), _i32),
            pltpu.VMEM((NB, EB), _i32),
            pltpu.VMEM((NB, EB), _i32),
            pltpu.VMEM((EB2, EB), _f32),
            pltpu.VMEM((EB2, EB), _f32),
            pltpu.VMEM_SHARED((NP, EB), _f32),
        ],
    )
    return fn(dst16_t, src16_t)


def _scatter_body(*refs):
    ys = refs[:NCHUNK]
    src_h, dst_h = refs[NCHUNK:NCHUNK + 2]
    zs = refs[NCHUNK + 2:2 * NCHUNK + 2]
    src_v, dst_v, gb0, gb1, acc, sem0, sem1 = refs[2 * NCHUNK + 2:]
    c = lax.axis_index("c")
    s = lax.axis_index("s")
    pltpu.sync_copy(src_h.at[s], src_v)
    pltpu.sync_copy(dst_h.at[s], dst_v)

    for p in range(PASSES):
        _fill_buf(gb0, EB2, 0.0)
        for k in range(RT // EB2):
            @pl.when(s * RT + (k + 1) * EB2 <= NP_ACC)
            def _(k=k):
                pltpu.sync_copy(gb0, acc.at[pl.ds(s * RT + k * EB2, EB2)])

        for cc in range(2):
            yc = ys[cc * PASSES + p]

            @pl.when(c == cc)
            def _(yc=yc):
                pltpu.async_copy(yc.at[src_v.at[0, pl.ds(0, EB2)]], gb0, sem0)

        plsc.subcore_barrier()

        for cc in range(2):
            yc = ys[cc * PASSES + p]

            @pl.when(c == cc)
            def _(yc=yc):
                lo = pl.ds(0, EB2)
                hi = pl.ds(EB2, EB2)

                def body(jj, carry):
                    pltpu.async_copy(yc.at[src_v.at[jj, hi]], gb1, sem1)
                    pltpu.make_async_copy(
                        yc.at[src_v.at[jj, lo]], gb0, sem0).wait()
                    pltpu.sync_copy(gb0, acc.at[dst_v.at[jj, lo]], add=True)

                    @pl.when(jj < NB - 1)
                    def _():
                        pltpu.async_copy(
                            yc.at[src_v.at[jj + 1, lo]], gb0, sem0)

                    pltpu.make_async_copy(
                        yc.at[src_v.at[jj, hi]], gb1, sem1).wait()
                    pltpu.sync_copy(gb1, acc.at[dst_v.at[jj, hi]], add=True)
                    return carry

                lax.fori_loop(0, NB, body, 0)

        plsc.subcore_barrier()

        for cc in range(2):
            zc = zs[cc * PASSES + p]

            @pl.when(c == cc)
            def _(zc=zc):
                for k in range(RT // EB2):
                    @pl.when(s * RT + (k + 1) * EB2 <= NP_ACC)
                    def _(k=k):
                        pltpu.sync_copy(acc.at[pl.ds(s * RT + k * EB2, EB2)], gb0)
                        pltpu.sync_copy(gb0, zc.at[pl.ds(s * RT + k * EB2, EB2)])

        if p != PASSES - 1:
            plsc.subcore_barrier()


def _sc_scatter(ychunks, srcw, dstw):
    fn = pl.kernel(
        _scatter_body,
        out_type=[jax.ShapeDtypeStruct((NP, CW), _f32)] * NCHUNK,
        mesh=_sc_mesh,
        scratch_types=[
            pltpu.VMEM((NB, EB), _i32),
            pltpu.VMEM((NB, EB), _i32),
            pltpu.VMEM((EB2, CW), _f32),
            pltpu.VMEM((EB2, CW), _f32),
            pltpu.VMEM_SHARED((NP_ACC, CW), _f32),
            pltpu.SemaphoreType.DMA,
            pltpu.SemaphoreType.DMA,
        ],
    )
    return fn(*ychunks, srcw, dstw)


def _dinv_block(dg_ref):
    d = dg_ref[0][:, 0:1] + dg_ref[1][:, 0:1]
    return lax.rsqrt(1.0 + d)


def _mm_scale_body(x_ref, w_ref, dg_ref, *outs):
    dinv = _dinv_block(dg_ref)
    y = jnp.dot(x_ref[...] * dinv, w_ref[...], preferred_element_type=_f32)
    for cidx, o in enumerate(outs):
        o[...] = y[:, cidx * CW:(cidx + 1) * CW]


def _mm_scale(x_p, W1, degpart):
    return pl.pallas_call(
        _mm_scale_body,
        grid=(NROWB,),
        in_specs=[
            pl.BlockSpec((RB, F_IN), lambda i: (i, 0)),
            pl.BlockSpec((F_IN, H), lambda i: (0, 0)),
            pl.BlockSpec((2, RB, EB), lambda i: (0, i, 0)),
        ],
        out_specs=[pl.BlockSpec((RB, CW), lambda i: (i, 0))] * NCHUNK,
        out_shape=[jax.ShapeDtypeStruct((NP, CW), _f32)] * NCHUNK,
    )(x_p, W1, degpart)


def _mid_body(*refs):
    zs = refs[:NCHUNK]
    ys = refs[NCHUNK:2 * NCHUNK]
    dg_ref, b_ref, w_ref = refs[2 * NCHUNK:2 * NCHUNK + 3]
    outs = refs[2 * NCHUNK + 3:]
    dinv = _dinv_block(dg_ref)
    hs = jnp.concatenate([zr[...] + yr[...] for zr, yr in zip(zs, ys)], axis=1)
    t = dinv * jax.nn.relu(dinv * hs + b_ref[...])
    out = jnp.dot(t, w_ref[...], preferred_element_type=_f32)
    for cidx, o in enumerate(outs):
        o[...] = out[:, cidx * CW:(cidx + 1) * CW]


def _mid_layer(zchunks, ychunks, degpart, b1r, W2):
    return pl.pallas_call(
        _mid_body,
        grid=(NROWB,),
        in_specs=(
            [pl.BlockSpec((RB, CW), lambda i: (i, 0))] * (2 * NCHUNK)
            + [
                pl.BlockSpec((2, RB, EB), lambda i: (0, i, 0)),
                pl.BlockSpec((1, H), lambda i: (0, 0)),
                pl.BlockSpec((H, H), lambda i: (0, 0)),
            ]
        ),
        out_specs=[pl.BlockSpec((RB, CW), lambda i: (i, 0))] * NCHUNK,
        out_shape=[jax.ShapeDtypeStruct((NP, CW), _f32)] * NCHUNK,
    )(*zchunks, *ychunks, degpart, b1r, W2)


def _final_body(*refs):
    zs = refs[:NCHUNK]
    ys = refs[NCHUNK:2 * NCHUNK]
    dg_ref, b_ref, bt_ref, wl_ref, bl_ref = refs[2 * NCHUNK:2 * NCHUNK + 5]
    hg_ref, lp_ref, cnt_ref = refs[2 * NCHUNK + 5:]
    i = pl.program_id(0)

    @pl.when(i == 0)
    def _():
        hg_ref[...] = jnp.zeros_like(hg_ref)
        cnt_ref[...] = jnp.zeros_like(cnt_ref)
        lp_ref[...] = jnp.zeros_like(lp_ref)

    dinv = _dinv_block(dg_ref)
    hs = jnp.concatenate([zr[...] + yr[...] for zr, yr in zip(zs, ys)], axis=1)
    bt = bt_ref[...]
    h = jax.nn.relu(dinv * hs + b_ref[...])
    h = jnp.where(bt < G, h, 0.0)
    gid = lax.broadcasted_iota(_i32, (1, G), 1)
    oh = (bt == gid).astype(_f32)
    dims = (((0,), (0,)), ((), ()))
    hg_ref[...] += lax.dot_general(oh, h, dims, preferred_element_type=_f32)
    cnt_ref[...] += lax.dot_general(oh, jnp.ones((RB, 128), _f32), dims,
                                    preferred_element_type=_f32)

    @pl.when(i == NROWB - 1)
    def _():
        cnt = jnp.maximum(cnt_ref[:, 0:1], 1.0)
        hg = hg_ref[...] / cnt
        hg_ref[...] = hg
        logits = jnp.dot(hg, wl_ref[...], preferred_element_type=_f32) + bl_ref[...]
        m = jnp.max(logits, axis=1, keepdims=True)
        lse = jnp.log(jnp.sum(jnp.exp(logits - m), axis=1, keepdims=True)) + m
        lp_ref[...] = logits - lse


def _final(zchunks, ychunks, degpart, b2r, batch_p, Wl, blr):
    return pl.pallas_call(
        _final_body,
        grid=(NROWB,),
        in_specs=(
            [pl.BlockSpec((RB, CW), lambda i: (i, 0))] * (2 * NCHUNK)
            + [
                pl.BlockSpec((2, RB, EB), lambda i: (0, i, 0)),
                pl.BlockSpec((1, H), lambda i: (0, 0)),
                pl.BlockSpec((RB, 1), lambda i: (i, 0)),
                pl.BlockSpec((H, C), lambda i: (0, 0)),
                pl.BlockSpec((1, C), lambda i: (0, 0)),
            ]
        ),
        out_specs=[
            pl.BlockSpec((G, H), lambda i: (0, 0)),
            pl.BlockSpec((G, C), lambda i: (0, 0)),
        ],
        out_shape=[
            jax.ShapeDtypeStruct((G, H), _f32),
            jax.ShapeDtypeStruct((G, C), _f32),
        ],
        scratch_shapes=[pltpu.VMEM((G, 128), _f32)],
        compiler_params=pltpu.CompilerParams(
            dimension_semantics=("arbitrary",)),
    )(*zchunks, *ychunks, degpart, b2r, batch_p, Wl, blr)


@jax.jit
def kernel(x, edge_index, batch, W1, b1, W2, b2, Wl, bl):
    src = edge_index[0].astype(_i32)
    dst = edge_index[1].astype(_i32)
    pad = jnp.full((E_PAD - E,), N, _i32)
    def _pack(a):
        a = jnp.concatenate([a, pad]).reshape(NTILE, NB // 2, 2, EB)
        return a[:, :, 0, :] | (a[:, :, 1, :] << 16)

    src16_t = _pack(src)
    dst16_t = _pack(dst)
    x_p = jnp.pad(x, ((0, NP - N), (0, 0)))
    batch_p = jnp.concatenate(
        [batch.astype(_i32), jnp.full((NP - N,), G, _i32)]).reshape(NP, 1)
    degpart, srcw, dstw = _deg_kernel(dst16_t, src16_t)
    y1 = _mm_scale(x_p, W1, degpart)
    zc1 = _sc_scatter(y1, srcw, dstw)
    y2 = _mid_layer(zc1, y1, degpart, b1.reshape(1, H), W2)
    zc2 = _sc_scatter(y2, srcw, dstw)
    hG, logp = _final(zc2, y2, degpart, b2.reshape(1, H), batch_p,
                      Wl, bl.reshape(1, C))
    return (hG, logp)


# speedup vs baseline: 1.1563x; 1.0273x over previous
import jax
import jax.numpy as jnp
from jax import lax
from jax.experimental import pallas as pl
from jax.experimental.pallas import tpu as pltpu
from jax.experimental.pallas import tpu_sc as plsc

N = 10000
E = 160000
F_IN = 256
H = 512
C = 10
G = 64

NP = 10240
RT = NP // 16
NTILE = 16
EB = 128
NB = 80
E_PAD = NTILE * NB * EB
EB2 = 64
NB2 = NB * 2
NP_ACC = 10048
NCHUNK = 4
CW = H // NCHUNK
PASSES = NCHUNK // 2
NROWB = 10
RB = NP // NROWB

_f32 = jnp.float32
_i32 = jnp.int32
_i16 = jnp.int16

_sc_mesh = plsc.VectorSubcoreMesh(core_axis_name="c", subcore_axis_name="s")


def _widen_idx(pk_h, pk_ref, idx32_ref):
    mask = jnp.full((16,), 0xFFFF, _i32)
    pltpu.sync_copy(pk_h, pk_ref)

    def body(j, carry):
        for k in range(EB // 16):
            v = pk_ref[j, pl.ds(k * 16, 16)]
            idx32_ref[2 * j, pl.ds(k * 16, 16)] = lax.bitwise_and(v, mask)
            idx32_ref[2 * j + 1, pl.ds(k * 16, 16)] = (
                lax.shift_right_logical(v, 16))
        return carry

    lax.fori_loop(0, NB // 2, body, 0)


def _fill_buf(zb_ref, nrows, val):
    v16 = jnp.full((16,), val, _f32)

    def body(r, carry):
        for k in range(CW // 16):
            zb_ref[r, pl.ds(k * 16, 16)] = v16
        return carry

    lax.fori_loop(0, nrows, body, 0)


def _deg_body(dst_h, src_h, out_h, srcw_h, dstw_h, d16, dst_v, src_v,
              ones_v, gb, acc):
    c = lax.axis_index("c")
    s = lax.axis_index("s")
    _widen_idx(dst_h.at[s], d16, dst_v)
    pltpu.sync_copy(dst_v, dstw_h.at[s])
    _widen_idx(src_h.at[s], d16, src_v)
    pltpu.sync_copy(src_v, srcw_h.at[s])
    _fill_buf(ones_v, EB2, 1.0)
    _fill_buf(gb, EB2, 0.0)
    for k in range(RT // EB2):
        pltpu.sync_copy(gb, acc.at[pl.ds(s * RT + k * EB2, EB2)])
    plsc.subcore_barrier()

    def body(j, carry):
        b = c * NB + j
        pltpu.sync_copy(
            ones_v, acc.at[dst_v.at[b // 2, pl.ds((b % 2) * EB2, EB2)]],
            add=True)
        return carry

    lax.fori_loop(0, NB, body, 0)
    plsc.subcore_barrier()
    for k in range(RT // EB2):
        pltpu.sync_copy(acc.at[pl.ds(s * RT + k * EB2, EB2)], gb)
        pltpu.sync_copy(gb, out_h.at[c, pl.ds(s * RT + k * EB2, EB2)])


def _deg_kernel(dst16_t, src16_t):
    fn = pl.kernel(
        _deg_body,
        out_type=[
            jax.ShapeDtypeStruct((2, NP, EB), _f32),
            jax.ShapeDtypeStruct((NTILE, NB, EB), _i32),
            jax.ShapeDtypeStruct((NTILE, NB, EB), _i32),
        ],
        mesh=_sc_mesh,
        scratch_types=[
            pltpu.VMEM((NB // 2, EB), _i32),
            pltpu.VMEM((NB, EB), _i32),
            pltpu.VMEM((NB, EB), _i32),
            pltpu.VMEM((EB2, EB), _f32),
            pltpu.VMEM((EB2, EB), _f32),
            pltpu.VMEM_SHARED((NP, EB), _f32),
        ],
    )
    return fn(dst16_t, src16_t)


def _scatter_body(*refs):
    ys = refs[:NCHUNK]
    src_h, dst_h = refs[NCHUNK:NCHUNK + 2]
    zs = refs[NCHUNK + 2:2 * NCHUNK + 2]
    (src_v, dst_v, gb0, gb1, gb2, gb3,
     acc, sem0, sem1, sem2, sem3) = refs[2 * NCHUNK + 2:]
    c = lax.axis_index("c")
    s = lax.axis_index("s")
    pltpu.sync_copy(src_h.at[s], src_v)
    pltpu.sync_copy(dst_h.at[s], dst_v)

    for p in range(PASSES):
        _fill_buf(gb0, 32, 0.0)
        for k in range(RT // 32):
            @pl.when(s * RT + (k + 1) * 32 <= NP_ACC)
            def _(k=k):
                pltpu.sync_copy(gb0, acc.at[pl.ds(s * RT + k * 32, 32)])

        for cc in range(2):
            yc = ys[cc * PASSES + p]

            @pl.when(c == cc)
            def _(yc=yc):
                for q, (gbq, semq) in enumerate(
                        zip((gb0, gb1, gb2, gb3), (sem0, sem1, sem2, sem3))):
                    pltpu.async_copy(
                        yc.at[src_v.at[0, pl.ds(q * 32, 32)]], gbq, semq)

        plsc.subcore_barrier()

        for cc in range(2):
            yc = ys[cc * PASSES + p]

            @pl.when(c == cc)
            def _(yc=yc):
                def body(jj, carry):
                    for q, (gbq, semq) in enumerate(
                            zip((gb0, gb1, gb2, gb3),
                                (sem0, sem1, sem2, sem3))):
                        sl = pl.ds(q * 32, 32)
                        pltpu.make_async_copy(
                            yc.at[src_v.at[jj, sl]], gbq, semq).wait()
                        pltpu.sync_copy(
                            gbq, acc.at[dst_v.at[jj, sl]], add=True)

                        @pl.when(jj < NB - 1)
                        def _(gbq=gbq, semq=semq, sl=sl):
                            pltpu.async_copy(
                                yc.at[src_v.at[jj + 1, sl]], gbq, semq)
                    return carry

                lax.fori_loop(0, NB, body, 0)

        plsc.subcore_barrier()

        for cc in range(2):
            zc = zs[cc * PASSES + p]

            @pl.when(c == cc)
            def _(zc=zc):
                for k in range(RT // EB2):
                    @pl.when(s * RT + (k + 1) * EB2 <= NP_ACC)
                    def _(k=k):
                        base = pl.ds(s * RT + k * EB2, 32)
                        base2 = pl.ds(s * RT + k * EB2 + 32, 32)
                        pltpu.sync_copy(acc.at[base], gb0)
                        pltpu.sync_copy(gb0, zc.at[base])
                        pltpu.sync_copy(acc.at[base2], gb1)
                        pltpu.sync_copy(gb1, zc.at[base2])

        if p != PASSES - 1:
            plsc.subcore_barrier()


def _sc_scatter(ychunks, srcw, dstw):
    fn = pl.kernel(
        _scatter_body,
        out_type=[jax.ShapeDtypeStruct((NP, CW), _f32)] * NCHUNK,
        mesh=_sc_mesh,
        scratch_types=[
            pltpu.VMEM((NB, EB), _i32),
            pltpu.VMEM((NB, EB), _i32),
            pltpu.VMEM((32, CW), _f32),
            pltpu.VMEM((32, CW), _f32),
            pltpu.VMEM((32, CW), _f32),
            pltpu.VMEM((32, CW), _f32),
            pltpu.VMEM_SHARED((NP_ACC, CW), _f32),
            pltpu.SemaphoreType.DMA,
            pltpu.SemaphoreType.DMA,
            pltpu.SemaphoreType.DMA,
            pltpu.SemaphoreType.DMA,
        ],
    )
    return fn(*ychunks, srcw, dstw)


def _dinv_block(dg_ref):
    d = dg_ref[0][:, 0:1] + dg_ref[1][:, 0:1]
    return lax.rsqrt(1.0 + d)


def _mm_scale_body(x_ref, w_ref, dg_ref, *outs):
    dinv = _dinv_block(dg_ref)
    y = jnp.dot(x_ref[...] * dinv, w_ref[...], preferred_element_type=_f32)
    for cidx, o in enumerate(outs):
        o[...] = y[:, cidx * CW:(cidx + 1) * CW]


def _mm_scale(x_p, W1, degpart):
    return pl.pallas_call(
        _mm_scale_body,
        grid=(NROWB,),
        in_specs=[
            pl.BlockSpec((RB, F_IN), lambda i: (i, 0)),
            pl.BlockSpec((F_IN, H), lambda i: (0, 0)),
            pl.BlockSpec((2, RB, EB), lambda i: (0, i, 0)),
        ],
        out_specs=[pl.BlockSpec((RB, CW), lambda i: (i, 0))] * NCHUNK,
        out_shape=[jax.ShapeDtypeStruct((NP, CW), _f32)] * NCHUNK,
    )(x_p, W1, degpart)


def _mid_body(*refs):
    zs = refs[:NCHUNK]
    ys = refs[NCHUNK:2 * NCHUNK]
    dg_ref, b_ref, w_ref = refs[2 * NCHUNK:2 * NCHUNK + 3]
    outs = refs[2 * NCHUNK + 3:]
    dinv = _dinv_block(dg_ref)
    hs = jnp.concatenate([zr[...] + yr[...] for zr, yr in zip(zs, ys)], axis=1)
    t = dinv * jax.nn.relu(dinv * hs + b_ref[...])
    out = jnp.dot(t, w_ref[...], preferred_element_type=_f32)
    for cidx, o in enumerate(outs):
        o[...] = out[:, cidx * CW:(cidx + 1) * CW]


def _mid_layer(zchunks, ychunks, degpart, b1r, W2):
    return pl.pallas_call(
        _mid_body,
        grid=(NROWB,),
        in_specs=(
            [pl.BlockSpec((RB, CW), lambda i: (i, 0))] * (2 * NCHUNK)
            + [
                pl.BlockSpec((2, RB, EB), lambda i: (0, i, 0)),
                pl.BlockSpec((1, H), lambda i: (0, 0)),
                pl.BlockSpec((H, H), lambda i: (0, 0)),
            ]
        ),
        out_specs=[pl.BlockSpec((RB, CW), lambda i: (i, 0))] * NCHUNK,
        out_shape=[jax.ShapeDtypeStruct((NP, CW), _f32)] * NCHUNK,
    )(*zchunks, *ychunks, degpart, b1r, W2)


def _final_body(*refs):
    zs = refs[:NCHUNK]
    ys = refs[NCHUNK:2 * NCHUNK]
    dg_ref, b_ref, bt_ref, wl_ref, bl_ref = refs[2 * NCHUNK:2 * NCHUNK + 5]
    hg_ref, lp_ref, cnt_ref = refs[2 * NCHUNK + 5:]
    i = pl.program_id(0)

    @pl.when(i == 0)
    def _():
        hg_ref[...] = jnp.zeros_like(hg_ref)
        cnt_ref[...] = jnp.zeros_like(cnt_ref)
        lp_ref[...] = jnp.zeros_like(lp_ref)

    dinv = _dinv_block(dg_ref)
    hs = jnp.concatenate([zr[...] + yr[...] for zr, yr in zip(zs, ys)], axis=1)
    bt = bt_ref[...]
    h = jax.nn.relu(dinv * hs + b_ref[...])
    h = jnp.where(bt < G, h, 0.0)
    gid = lax.broadcasted_iota(_i32, (1, G), 1)
    oh = (bt == gid).astype(_f32)
    dims = (((0,), (0,)), ((), ()))
    hg_ref[...] += lax.dot_general(oh, h, dims, preferred_element_type=_f32)
    cnt_ref[...] += lax.dot_general(oh, jnp.ones((RB, 128), _f32), dims,
                                    preferred_element_type=_f32)

    @pl.when(i == NROWB - 1)
    def _():
        cnt = jnp.maximum(cnt_ref[:, 0:1], 1.0)
        hg = hg_ref[...] / cnt
        hg_ref[...] = hg
        logits = jnp.dot(hg, wl_ref[...], preferred_element_type=_f32) + bl_ref[...]
        m = jnp.max(logits, axis=1, keepdims=True)
        lse = jnp.log(jnp.sum(jnp.exp(logits - m), axis=1, keepdims=True)) + m
        lp_ref[...] = logits - lse


def _final(zchunks, ychunks, degpart, b2r, batch_p, Wl, blr):
    return pl.pallas_call(
        _final_body,
        grid=(NROWB,),
        in_specs=(
            [pl.BlockSpec((RB, CW), lambda i: (i, 0))] * (2 * NCHUNK)
            + [
                pl.BlockSpec((2, RB, EB), lambda i: (0, i, 0)),
                pl.BlockSpec((1, H), lambda i: (0, 0)),
                pl.BlockSpec((RB, 1), lambda i: (i, 0)),
                pl.BlockSpec((H, C), lambda i: (0, 0)),
                pl.BlockSpec((1, C), lambda i: (0, 0)),
            ]
        ),
        out_specs=[
            pl.BlockSpec((G, H), lambda i: (0, 0)),
            pl.BlockSpec((G, C), lambda i: (0, 0)),
        ],
        out_shape=[
            jax.ShapeDtypeStruct((G, H), _f32),
            jax.ShapeDtypeStruct((G, C), _f32),
        ],
        scratch_shapes=[pltpu.VMEM((G, 128), _f32)],
        compiler_params=pltpu.CompilerParams(
            dimension_semantics=("arbitrary",)),
    )(*zchunks, *ychunks, degpart, b2r, batch_p, Wl, blr)


@jax.jit
def kernel(x, edge_index, batch, W1, b1, W2, b2, Wl, bl):
    src = edge_index[0].astype(_i32)
    dst = edge_index[1].astype(_i32)
    pad = jnp.full((E_PAD - E,), N, _i32)
    def _pack(a):
        a = jnp.concatenate([a, pad]).reshape(NTILE, NB // 2, 2, EB)
        return a[:, :, 0, :] | (a[:, :, 1, :] << 16)

    src16_t = _pack(src)
    dst16_t = _pack(dst)
    x_p = jnp.pad(x, ((0, NP - N), (0, 0)))
    batch_p = jnp.concatenate(
        [batch.astype(_i32), jnp.full((NP - N,), G, _i32)]).reshape(NP, 1)
    degpart, srcw, dstw = _deg_kernel(dst16_t, src16_t)
    y1 = _mm_scale(x_p, W1, degpart)
    zc1 = _sc_scatter(y1, srcw, dstw)
    y2 = _mid_layer(zc1, y1, degpart, b1.reshape(1, H), W2)
    zc2 = _sc_scatter(y2, srcw, dstw)
    hG, logp = _final(zc2, y2, degpart, b2.reshape(1, H), batch_p,
                      Wl, bl.reshape(1, C))
    return (hG, logp)
